# baseline (speedup 1.0000x reference)
import functools

import jax
import jax.numpy as jnp
from jax import lax
from jax.experimental import pallas as pl
from jax.experimental.pallas import tpu as pltpu
from jax.experimental.pallas import tpu_sc as plsc

N = 10000
E = 160000
F_IN = 128
DIM = 32
B = 128
D_EDGE = 5

NC = 2
NS = 16
NW = NC * NS
CH = 128
NCHUNK_W = 40
ROWS_W = CH * NCHUNK_W
EPAD = NW * ROWS_W
WIDTH = DIM + 16
NBLK_E = EPAD // 1024
NBLK_N = 10
ROWS_S = N // NS


def _in_mlp_body(x_ref, w_ref, b_ref, o_ref):
    o_ref[...] = jax.nn.relu(
        jnp.dot(x_ref[...], w_ref[...], preferred_element_type=jnp.float32)
        + b_ref[...])


def _t_body(ea_ref, w1_ref, b1_ref, o_ref):
    o_ref[...] = jax.nn.relu(
        jnp.dot(ea_ref[...], w1_ref[...], preferred_element_type=jnp.float32)
        + b1_ref[...]).astype(jnp.bfloat16)


def _msg_body(xj_ref, t_ref, w2_ref, b2_ref, r_ref, s_ref, o_ref):
    i = pl.program_id(0)
    ew = (jnp.dot(t_ref[...], w2_ref[...],
                  preferred_element_type=jnp.float32)
          + b2_ref[...]).astype(jnp.bfloat16)
    xe = jnp.dot(xj_ref[...].astype(jnp.bfloat16), r_ref[...],
                 preferred_element_type=jnp.float32)
    p = xe.astype(jnp.bfloat16) * ew
    msg = jnp.dot(p, s_ref[...], preferred_element_type=jnp.float32)
    rows = i * 1024 + lax.broadcasted_iota(jnp.int32, (1024, 1), 0)
    m = (rows < E).astype(jnp.float32)
    o_ref[...] = jnp.concatenate(
        [msg * m, jnp.broadcast_to(m, (1024, 16))], axis=1)


def _gru_body(p_ref, h_ref, wih_ref, whh_ref, bih_ref, bhh_ref, bc_ref,
              o_ref):
    s = p_ref[0] + p_ref[1]
    ssum = s[:, :DIM]
    cnt = s[:, DIM:DIM + 1]
    h = h_ref[...]
    aggr = ssum / jnp.maximum(cnt, 1.0) + bc_ref[...]
    m = jax.nn.relu(aggr)
    gi = jnp.dot(m, wih_ref[...], preferred_element_type=jnp.float32) \
        + bih_ref[...]
    gh = jnp.dot(h, whh_ref[...], preferred_element_type=jnp.float32) \
        + bhh_ref[...]
    r = jax.nn.sigmoid(gi[:, :DIM] + gh[:, :DIM])
    z = jax.nn.sigmoid(gi[:, DIM:2 * DIM] + gh[:, DIM:2 * DIM])
    n = jnp.tanh(gi[:, 2 * DIM:] + r * gh[:, 2 * DIM:])
    o_ref[...] = (1.0 - z) * n + z * h


def _s2s_body(out_ref, batch_ref, wi_ref, wh_ref, bi_ref, bh_ref, q_ref):
    outv = out_ref[...]
    bI = lax.broadcasted_iota(jnp.int32, (B, N), 0)
    Mb = jnp.broadcast_to(batch_ref[...], (B, N)) == bI
    qs = jnp.zeros((B, 2 * DIM), jnp.float32)
    hs = jnp.zeros((B, DIM), jnp.float32)
    cs = jnp.zeros((B, DIM), jnp.float32)
    for _ in range(3):
        g = (jnp.dot(qs, wi_ref[...], preferred_element_type=jnp.float32)
             + bi_ref[...]
             + jnp.dot(hs, wh_ref[...], preferred_element_type=jnp.float32)
             + bh_ref[...])
        ig = jax.nn.sigmoid(g[:, :DIM])
        fg = jax.nn.sigmoid(g[:, DIM:2 * DIM])
        gg = jnp.tanh(g[:, 2 * DIM:3 * DIM])
        og = jax.nn.sigmoid(g[:, 3 * DIM:])
        cs = fg * cs + ig * gg
        hs = og * jnp.tanh(cs)
        sT = lax.dot_general(hs, outv, (((1,), (1,)), ((), ())),
                             preferred_element_type=jnp.float32)
        emax = jnp.max(jnp.where(Mb, sT, -1e30), axis=1, keepdims=True)
        a = jnp.where(Mb, jnp.exp(sT - emax), 0.0)
        denom = jnp.maximum(jnp.sum(a, axis=1, keepdims=True), 1e-30)
        an = a / denom
        r = jnp.dot(an, outv, preferred_element_type=jnp.float32)
        qs = jnp.concatenate([hs, r], axis=1)
    q_ref[...] = qs


@functools.cache
def _sc_kernels():
    mesh = plsc.VectorSubcoreMesh(core_axis_name="c", subcore_axis_name="s",
                                  num_cores=NC, num_subcores=NS)

    params = pltpu.CompilerParams(use_tc_tiling_on_sc=False)

    @functools.partial(
        pl.kernel, mesh=mesh, compiler_params=params,
        out_type=jax.ShapeDtypeStruct((EPAD, DIM), jnp.float32),
        scratch_types=[
            pltpu.VMEM((NCHUNK_W, CH), jnp.int32),
            pltpu.VMEM((2, CH, DIM), jnp.float32),
            pltpu.SemaphoreType.DMA,
        ])
    def sc_gather(h_hbm, src_hbm, xj_hbm, idx_v, rows_v, sem):
        w = lax.axis_index("s") * NC + lax.axis_index("c")
        pltpu.sync_copy(src_hbm.at[pl.ds(w * NCHUNK_W, NCHUNK_W)], idx_v)
        pltpu.async_copy(h_hbm.at[idx_v.at[0]], rows_v.at[0], sem)

        def body(j, carry):
            @pl.when(j + 1 < NCHUNK_W)
            def _():
                pltpu.async_copy(h_hbm.at[idx_v.at[j + 1]],
                                 rows_v.at[(j + 1) % 2], sem)
            pltpu.make_async_copy(h_hbm.at[idx_v.at[j]],
                                  rows_v.at[j % 2], sem).wait()
            pltpu.sync_copy(rows_v.at[j % 2],
                            xj_hbm.at[pl.ds(w * ROWS_W + j * CH, CH)])
            return carry

        lax.fori_loop(0, NCHUNK_W, body, 0)

    @functools.partial(
        pl.kernel, mesh=mesh, compiler_params=params,
        out_type=jax.ShapeDtypeStruct((NC, N, WIDTH), jnp.float32),
        scratch_types=[
            pltpu.VMEM((NCHUNK_W, CH), jnp.int32),
            pltpu.VMEM((2, CH, WIDTH), jnp.float32),
            pltpu.VMEM_SHARED((N, WIDTH), jnp.float32),
            pltpu.SemaphoreType.DMA,
        ])
    def sc_scatter(msg_hbm, dst_hbm, zeros_hbm, out_hbm, idx_v, vals_v,
                   acc_sh, sem):
        c = lax.axis_index("c")
        s = lax.axis_index("s")
        w = s * NC + c
        pltpu.sync_copy(zeros_hbm, acc_sh.at[pl.ds(s * ROWS_S, ROWS_S)])
        plsc.subcore_barrier()
        pltpu.sync_copy(dst_hbm.at[pl.ds(w * NCHUNK_W, NCHUNK_W)], idx_v)
        pltpu.async_copy(msg_hbm.at[pl.ds(w * ROWS_W, CH)], vals_v.at[0],
                         sem)

        def body(j, carry):
            @pl.when(j + 1 < NCHUNK_W)
            def _():
                pltpu.async_copy(
                    msg_hbm.at[pl.ds(w * ROWS_W + (j + 1) * CH, CH)],
                    vals_v.at[(j + 1) % 2], sem)
            pltpu.make_async_copy(
                msg_hbm.at[pl.ds(w * ROWS_W + j * CH, CH)],
                vals_v.at[j % 2], sem).wait()
            pltpu.sync_copy(vals_v.at[j % 2], acc_sh.at[idx_v.at[j]],
                            add=True)
            return carry

        lax.fori_loop(0, NCHUNK_W, body, 0)
        plsc.subcore_barrier()
        pltpu.sync_copy(acc_sh.at[pl.ds(s * ROWS_S, ROWS_S)],
                        out_hbm.at[c, pl.ds(s * ROWS_S, ROWS_S)])

    return sc_gather, sc_scatter


def _tc(body, grid, in_specs, out_specs, out_shape):
    return pl.pallas_call(body, grid=grid, in_specs=in_specs,
                          out_specs=out_specs, out_shape=out_shape)


def kernel(x, edge_index, edge_attr, batch, W0, b0, We1, be1, We2, be2,
           bconv, Wih, Whh, bih, bhh, Ws_ih, Ws_hh, bs_ih, bs_hh):
    f32 = jnp.float32
    src = edge_index[0]
    dst = edge_index[1]
    pad = EPAD - E
    src_p = jnp.concatenate([src, jnp.zeros((pad,), jnp.int32)]) \
        .reshape(EPAD // CH, CH)
    dst_p = jnp.concatenate([dst, jnp.zeros((pad,), jnp.int32)]) \
        .reshape(EPAD // CH, CH)
    ea_p = jnp.concatenate([edge_attr, jnp.zeros((pad, D_EDGE), f32)])

    cidx = jnp.arange(DIM * DIM, dtype=jnp.int32)
    R = (cidx[None, :] // DIM
         == jnp.arange(DIM, dtype=jnp.int32)[:, None]).astype(jnp.bfloat16)
    S = (cidx[:, None] % DIM
         == jnp.arange(DIM, dtype=jnp.int32)[None, :]).astype(jnp.bfloat16)
    zeros625 = jnp.zeros((ROWS_S, WIDTH), f32)

    full = lambda shp: pl.BlockSpec(shp, lambda: (0,) * len(shp))

    h = _tc(_in_mlp_body, (NBLK_N,),
            [pl.BlockSpec((N // NBLK_N, F_IN), lambda i: (i, 0)),
             pl.BlockSpec((F_IN, DIM), lambda i: (0, 0)),
             pl.BlockSpec((1, DIM), lambda i: (0, 0))],
            pl.BlockSpec((N // NBLK_N, DIM), lambda i: (i, 0)),
            jax.ShapeDtypeStruct((N, DIM), f32))(x, W0.T, b0[None])

    t = _tc(_t_body, (NBLK_E,),
            [pl.BlockSpec((1024, D_EDGE), lambda i: (i, 0)),
             pl.BlockSpec((D_EDGE, F_IN), lambda i: (0, 0)),
             pl.BlockSpec((1, F_IN), lambda i: (0, 0))],
            pl.BlockSpec((1024, F_IN), lambda i: (i, 0)),
            jax.ShapeDtypeStruct((EPAD, F_IN), jnp.bfloat16))(
                ea_p, We1.T, be1[None])

    msg_call = _tc(_msg_body, (NBLK_E,),
                   [pl.BlockSpec((1024, DIM), lambda i: (i, 0)),
                    pl.BlockSpec((1024, F_IN), lambda i: (i, 0)),
                    pl.BlockSpec((F_IN, DIM * DIM), lambda i: (0, 0)),
                    pl.BlockSpec((1, DIM * DIM), lambda i: (0, 0)),
                    pl.BlockSpec((DIM, DIM * DIM), lambda i: (0, 0)),
                    pl.BlockSpec((DIM * DIM, DIM), lambda i: (0, 0))],
                   pl.BlockSpec((1024, WIDTH), lambda i: (i, 0)),
                   jax.ShapeDtypeStruct((EPAD, WIDTH), f32))

    gru_call = _tc(_gru_body, (NBLK_N,),
                   [pl.BlockSpec((NC, N // NBLK_N, WIDTH),
                                 lambda i: (0, i, 0)),
                    pl.BlockSpec((N // NBLK_N, DIM), lambda i: (i, 0)),
                    pl.BlockSpec((DIM, 3 * DIM), lambda i: (0, 0)),
                    pl.BlockSpec((DIM, 3 * DIM), lambda i: (0, 0)),
                    pl.BlockSpec((1, 3 * DIM), lambda i: (0, 0)),
                    pl.BlockSpec((1, 3 * DIM), lambda i: (0, 0)),
                    pl.BlockSpec((1, DIM), lambda i: (0, 0))],
                   pl.BlockSpec((N // NBLK_N, DIM), lambda i: (i, 0)),
                   jax.ShapeDtypeStruct((N, DIM), f32))

    sc_gather, sc_scatter = _sc_kernels()
    for _ in range(3):
        xj = sc_gather(h, src_p)
        msga = msg_call(xj, t, We2.T.astype(jnp.bfloat16), be2[None], R, S)
        parts = sc_scatter(msga, dst_p, zeros625)
        h = gru_call(parts, h, Wih.T, Whh.T, bih[None], bhh[None],
                     bconv[None])

    qstar = _tc(_s2s_body, (),
                [full((N, DIM)), full((1, N)),
                 full((2 * DIM, 4 * DIM)), full((DIM, 4 * DIM)),
                 full((1, 4 * DIM)), full((1, 4 * DIM))],
                full((B, 2 * DIM)),
                jax.ShapeDtypeStruct((B, 2 * DIM), f32))(
                    h, batch[None], Ws_ih.T, Ws_hh.T, bs_ih[None],
                    bs_hh[None])
    return (qstar, h)


# speedup vs baseline: 1.0787x; 1.0787x over previous
import functools

import jax
import jax.numpy as jnp
from jax import lax
from jax.experimental import pallas as pl
from jax.experimental.pallas import tpu as pltpu
from jax.experimental.pallas import tpu_sc as plsc

N = 10000
E = 160000
F_IN = 128
DIM = 32
B = 128
D_EDGE = 5

NC = 2
NS = 16
NW = NC * NS
CH = 128
EH = E // 2
NCHUNK_W = 20
ROWS_W = CH * NCHUNK_W
EPAD = NW * ROWS_W
WIDTH = DIM + 16
NBLK_E = EPAD // 1024
NBLK_N = 10
ROWS_S = N // NS


def _in_mlp_body(x_ref, w_ref, b_ref, o_ref):
    o_ref[...] = jax.nn.relu(
        jnp.dot(x_ref[...], w_ref[...], preferred_element_type=jnp.float32)
        + b_ref[...])


def _ew_body(ea_ref, w1_ref, b1_ref, w2_ref, b2_ref, o_ref):
    t = jax.nn.relu(
        jnp.dot(ea_ref[...], w1_ref[...], preferred_element_type=jnp.float32)
        + b1_ref[...]).astype(jnp.bfloat16)
    o_ref[...] = (jnp.dot(t, w2_ref[...], preferred_element_type=jnp.float32)
                  + b2_ref[...]).astype(jnp.bfloat16)


def _msg_body(xj_ref, ew_ref, r_ref, s_ref, o_ref):
    i = pl.program_id(0)
    xe = jnp.dot(xj_ref[...].astype(jnp.bfloat16), r_ref[...],
                 preferred_element_type=jnp.float32)
    p = xe.astype(jnp.bfloat16) * ew_ref[...]
    msg = jnp.dot(p, s_ref[...], preferred_element_type=jnp.float32)
    rows = i * 1024 + lax.broadcasted_iota(jnp.int32, (1024, 1), 0)
    m = (rows < EH).astype(jnp.float32)
    o_ref[...] = jnp.concatenate(
        [msg * m, jnp.broadcast_to(m, (1024, 16))], axis=1)


def _gru_body(pa_ref, pb_ref, h_ref, wih_ref, whh_ref, bih_ref, bhh_ref,
              bc_ref, o_ref):
    s = pa_ref[0] + pa_ref[1] + pb_ref[0] + pb_ref[1]
    ssum = s[:, :DIM]
    cnt = s[:, DIM:DIM + 1]
    h = h_ref[...]
    aggr = ssum / jnp.maximum(cnt, 1.0) + bc_ref[...]
    m = jax.nn.relu(aggr)
    gi = jnp.dot(m, wih_ref[...], preferred_element_type=jnp.float32) \
        + bih_ref[...]
    gh = jnp.dot(h, whh_ref[...], preferred_element_type=jnp.float32) \
        + bhh_ref[...]
    r = jax.nn.sigmoid(gi[:, :DIM] + gh[:, :DIM])
    z = jax.nn.sigmoid(gi[:, DIM:2 * DIM] + gh[:, DIM:2 * DIM])
    n = jnp.tanh(gi[:, 2 * DIM:] + r * gh[:, 2 * DIM:])
    o_ref[...] = (1.0 - z) * n + z * h


def _s2s_body(out_ref, batch_ref, wi_ref, wh_ref, bi_ref, bh_ref, q_ref):
    outv = out_ref[...]
    bI = lax.broadcasted_iota(jnp.int32, (B, N), 0)
    Mb = jnp.broadcast_to(batch_ref[...], (B, N)) == bI
    qs = jnp.zeros((B, 2 * DIM), jnp.float32)
    hs = jnp.zeros((B, DIM), jnp.float32)
    cs = jnp.zeros((B, DIM), jnp.float32)
    for _ in range(3):
        g = (jnp.dot(qs, wi_ref[...], preferred_element_type=jnp.float32)
             + bi_ref[...]
             + jnp.dot(hs, wh_ref[...], preferred_element_type=jnp.float32)
             + bh_ref[...])
        ig = jax.nn.sigmoid(g[:, :DIM])
        fg = jax.nn.sigmoid(g[:, DIM:2 * DIM])
        gg = jnp.tanh(g[:, 2 * DIM:3 * DIM])
        og = jax.nn.sigmoid(g[:, 3 * DIM:])
        cs = fg * cs + ig * gg
        hs = og * jnp.tanh(cs)
        sT = lax.dot_general(hs, outv, (((1,), (1,)), ((), ())),
                             preferred_element_type=jnp.float32)
        emax = jnp.max(jnp.where(Mb, sT, -1e30), axis=1, keepdims=True)
        a = jnp.where(Mb, jnp.exp(sT - emax), 0.0)
        denom = jnp.maximum(jnp.sum(a, axis=1, keepdims=True), 1e-30)
        an = a / denom
        r = jnp.dot(an, outv, preferred_element_type=jnp.float32)
        qs = jnp.concatenate([hs, r], axis=1)
    q_ref[...] = qs


@functools.cache
def _sc_kernels():
    mesh = plsc.VectorSubcoreMesh(core_axis_name="c", subcore_axis_name="s",
                                  num_cores=NC, num_subcores=NS)
    params = pltpu.CompilerParams(use_tc_tiling_on_sc=False)

    @functools.partial(
        pl.kernel, mesh=mesh, compiler_params=params,
        out_type=jax.ShapeDtypeStruct((EPAD, DIM), jnp.float32),
        scratch_types=[
            pltpu.VMEM((NCHUNK_W, CH), jnp.int32),
            pltpu.VMEM((2, CH, DIM), jnp.float32),
            pltpu.SemaphoreType.DMA,
        ])
    def sc_gather(h_hbm, src_hbm, xj_hbm, idx_v, rows_v, sem):
        w = lax.axis_index("s") * NC + lax.axis_index("c")
        pltpu.sync_copy(src_hbm.at[pl.ds(w * NCHUNK_W, NCHUNK_W)], idx_v)
        pltpu.async_copy(h_hbm.at[idx_v.at[0]], rows_v.at[0], sem)

        def body(j, carry):
            @pl.when(j + 1 < NCHUNK_W)
            def _():
                pltpu.async_copy(h_hbm.at[idx_v.at[j + 1]],
                                 rows_v.at[(j + 1) % 2], sem)
            pltpu.make_async_copy(h_hbm.at[idx_v.at[j]],
                                  rows_v.at[j % 2], sem).wait()
            pltpu.sync_copy(rows_v.at[j % 2],
                            xj_hbm.at[pl.ds(w * ROWS_W + j * CH, CH)])
            return carry

        lax.fori_loop(0, NCHUNK_W, body, 0)

    @functools.partial(
        pl.kernel, mesh=mesh, compiler_params=params,
        out_type=jax.ShapeDtypeStruct((NC, N, WIDTH), jnp.float32),
        scratch_types=[
            pltpu.VMEM((NCHUNK_W, CH), jnp.int32),
            pltpu.VMEM((2, CH, WIDTH), jnp.float32),
            pltpu.VMEM_SHARED((N, WIDTH), jnp.float32),
            pltpu.SemaphoreType.DMA,
        ])
    def sc_scatter(msg_hbm, dst_hbm, zeros_hbm, out_hbm, idx_v, vals_v,
                   acc_sh, sem):
        c = lax.axis_index("c")
        s = lax.axis_index("s")
        w = s * NC + c
        pltpu.sync_copy(zeros_hbm, acc_sh.at[pl.ds(s * ROWS_S, ROWS_S)])
        plsc.subcore_barrier()
        pltpu.sync_copy(dst_hbm.at[pl.ds(w * NCHUNK_W, NCHUNK_W)], idx_v)
        pltpu.async_copy(msg_hbm.at[pl.ds(w * ROWS_W, CH)], vals_v.at[0],
                         sem)

        def body(j, carry):
            @pl.when(j + 1 < NCHUNK_W)
            def _():
                pltpu.async_copy(
                    msg_hbm.at[pl.ds(w * ROWS_W + (j + 1) * CH, CH)],
                    vals_v.at[(j + 1) % 2], sem)
            pltpu.make_async_copy(
                msg_hbm.at[pl.ds(w * ROWS_W + j * CH, CH)],
                vals_v.at[j % 2], sem).wait()
            pltpu.sync_copy(vals_v.at[j % 2], acc_sh.at[idx_v.at[j]],
                            add=True)
            return carry

        lax.fori_loop(0, NCHUNK_W, body, 0)
        plsc.subcore_barrier()
        pltpu.sync_copy(acc_sh.at[pl.ds(s * ROWS_S, ROWS_S)],
                        out_hbm.at[c, pl.ds(s * ROWS_S, ROWS_S)])

    return sc_gather, sc_scatter


def _tc(body, grid, in_specs, out_specs, out_shape):
    return pl.pallas_call(body, grid=grid, in_specs=in_specs,
                          out_specs=out_specs, out_shape=out_shape)


def _pad_half(v, width=None):
    pad = EPAD - EH
    if width is None:
        z = jnp.zeros((pad,), v.dtype)
        return jnp.concatenate([v, z]).reshape(EPAD // CH, CH)
    z = jnp.zeros((pad, width), v.dtype)
    return jnp.concatenate([v, z])


def kernel(x, edge_index, edge_attr, batch, W0, b0, We1, be1, We2, be2,
           bconv, Wih, Whh, bih, bhh, Ws_ih, Ws_hh, bs_ih, bs_hh):
    f32 = jnp.float32
    src = edge_index[0]
    dst = edge_index[1]
    srcs = [_pad_half(src[:EH]), _pad_half(src[EH:])]
    dsts = [_pad_half(dst[:EH]), _pad_half(dst[EH:])]
    eas = [_pad_half(edge_attr[:EH], D_EDGE),
           _pad_half(edge_attr[EH:], D_EDGE)]

    cidx = jnp.arange(DIM * DIM, dtype=jnp.int32)
    R = (cidx[None, :] // DIM
         == jnp.arange(DIM, dtype=jnp.int32)[:, None]).astype(jnp.bfloat16)
    S = (cidx[:, None] % DIM
         == jnp.arange(DIM, dtype=jnp.int32)[None, :]).astype(jnp.bfloat16)
    zeros625 = jnp.zeros((ROWS_S, WIDTH), f32)

    full = lambda shp: pl.BlockSpec(shp, lambda: (0,) * len(shp))

    h = _tc(_in_mlp_body, (NBLK_N,),
            [pl.BlockSpec((N // NBLK_N, F_IN), lambda i: (i, 0)),
             pl.BlockSpec((F_IN, DIM), lambda i: (0, 0)),
             pl.BlockSpec((1, DIM), lambda i: (0, 0))],
            pl.BlockSpec((N // NBLK_N, DIM), lambda i: (i, 0)),
            jax.ShapeDtypeStruct((N, DIM), f32))(x, W0.T, b0[None])

    ew_call = _tc(_ew_body, (NBLK_E,),
                  [pl.BlockSpec((1024, D_EDGE), lambda i: (i, 0)),
                   pl.BlockSpec((D_EDGE, F_IN), lambda i: (0, 0)),
                   pl.BlockSpec((1, F_IN), lambda i: (0, 0)),
                   pl.BlockSpec((F_IN, DIM * DIM), lambda i: (0, 0)),
                   pl.BlockSpec((1, DIM * DIM), lambda i: (0, 0))],
                  pl.BlockSpec((1024, DIM * DIM), lambda i: (i, 0)),
                  jax.ShapeDtypeStruct((EPAD, DIM * DIM), jnp.bfloat16))
    We1T = We1.T
    We2T = We2.T.astype(jnp.bfloat16)
    ews = [ew_call(eas[0], We1T, be1[None], We2T, be2[None]),
           ew_call(eas[1], We1T, be1[None], We2T, be2[None])]

    msg_call = _tc(_msg_body, (NBLK_E,),
                   [pl.BlockSpec((1024, DIM), lambda i: (i, 0)),
                    pl.BlockSpec((1024, DIM * DIM), lambda i: (i, 0)),
                    pl.BlockSpec((DIM, DIM * DIM), lambda i: (0, 0)),
                    pl.BlockSpec((DIM * DIM, DIM), lambda i: (0, 0))],
                   pl.BlockSpec((1024, WIDTH), lambda i: (i, 0)),
                   jax.ShapeDtypeStruct((EPAD, WIDTH), f32))

    gru_call = _tc(_gru_body, (NBLK_N,),
                   [pl.BlockSpec((NC, N // NBLK_N, WIDTH),
                                 lambda i: (0, i, 0)),
                    pl.BlockSpec((NC, N // NBLK_N, WIDTH),
                                 lambda i: (0, i, 0)),
                    pl.BlockSpec((N // NBLK_N, DIM), lambda i: (i, 0)),
                    pl.BlockSpec((DIM, 3 * DIM), lambda i: (0, 0)),
                    pl.BlockSpec((DIM, 3 * DIM), lambda i: (0, 0)),
                    pl.BlockSpec((1, 3 * DIM), lambda i: (0, 0)),
                    pl.BlockSpec((1, 3 * DIM), lambda i: (0, 0)),
                    pl.BlockSpec((1, DIM), lambda i: (0, 0))],
                   pl.BlockSpec((N // NBLK_N, DIM), lambda i: (i, 0)),
                   jax.ShapeDtypeStruct((N, DIM), f32))

    sc_gather, sc_scatter = _sc_kernels()
    WihT = Wih.T
    WhhT = Whh.T
    for _ in range(3):
        xj0 = sc_gather(h, srcs[0])
        msg0 = msg_call(xj0, ews[0], R, S)
        xj1 = sc_gather(h, srcs[1])
        parts0 = sc_scatter(msg0, dsts[0], zeros625)
        msg1 = msg_call(xj1, ews[1], R, S)
        parts1 = sc_scatter(msg1, dsts[1], zeros625)
        h = gru_call(parts0, parts1, h, WihT, WhhT, bih[None], bhh[None],
                     bconv[None])

    qstar = _tc(_s2s_body, (),
                [full((N, DIM)), full((1, N)),
                 full((2 * DIM, 4 * DIM)), full((DIM, 4 * DIM)),
                 full((1, 4 * DIM)), full((1, 4 * DIM))],
                full((B, 2 * DIM)),
                jax.ShapeDtypeStruct((B, 2 * DIM), f32))(
                    h, batch[None], Ws_ih.T, Ws_hh.T, bs_ih[None],
                    bs_hh[None])
    return (qstar, h)


# speedup vs baseline: 1.1971x; 1.1097x over previous
import functools

import jax
import jax.numpy as jnp
from jax import lax
from jax.experimental import pallas as pl
from jax.experimental.pallas import tpu as pltpu
from jax.experimental.pallas import tpu_sc as plsc

N = 10000
E = 160000
F_IN = 128
DIM = 32
B = 128
D_EDGE = 5

NC = 2
NS = 16
NW = NC * NS
CH = 128
EH = E // 2
NCHUNK_W = 20
ROWS_W = CH * NCHUNK_W
EPAD = NW * ROWS_W
WIDTH = DIM + 16
EBLK = 2048
NBLK_E = EPAD // EBLK
NBLK_N = 10
ROWS_S = N // NS


def _in_mlp_body(x_ref, w_ref, b_ref, o_ref):
    o_ref[...] = jax.nn.relu(
        jnp.dot(x_ref[...], w_ref[...], preferred_element_type=jnp.float32)
        + b_ref[...])


def _ew_body(ea_ref, w1_ref, b1_ref, w2_ref, b2_ref, o_ref):
    t = jax.nn.relu(
        jnp.dot(ea_ref[...], w1_ref[...], preferred_element_type=jnp.float32)
        + b1_ref[...]).astype(jnp.bfloat16)
    o_ref[...] = (jnp.dot(t, w2_ref[...], preferred_element_type=jnp.float32)
                  + b2_ref[...]).astype(jnp.bfloat16)


def _msg_body(xj_ref, ew_ref, r_ref, s_ref, o_ref):
    i = pl.program_id(0)
    xe = jnp.dot(xj_ref[...].astype(jnp.bfloat16), r_ref[...],
                 preferred_element_type=jnp.float32)
    p = xe.astype(jnp.bfloat16) * ew_ref[...]
    msg = jnp.dot(p, s_ref[...], preferred_element_type=jnp.float32)
    rows = i * EBLK + lax.broadcasted_iota(jnp.int32, (EBLK, 1), 0)
    m = (rows < EH).astype(jnp.float32)
    o_ref[...] = jnp.concatenate(
        [msg * m, jnp.broadcast_to(m, (EBLK, 16))], axis=1)


def _gru_body(pa_ref, pb_ref, h_ref, wih_ref, whh_ref, bih_ref, bhh_ref,
              bc_ref, o_ref):
    s = pa_ref[0] + pa_ref[1] + pb_ref[0] + pb_ref[1]
    ssum = s[:, :DIM]
    cnt = s[:, DIM:DIM + 1]
    h = h_ref[...]
    aggr = ssum / jnp.maximum(cnt, 1.0) + bc_ref[...]
    m = jax.nn.relu(aggr)
    gi = jnp.dot(m, wih_ref[...], preferred_element_type=jnp.float32) \
        + bih_ref[...]
    gh = jnp.dot(h, whh_ref[...], preferred_element_type=jnp.float32) \
        + bhh_ref[...]
    r = jax.nn.sigmoid(gi[:, :DIM] + gh[:, :DIM])
    z = jax.nn.sigmoid(gi[:, DIM:2 * DIM] + gh[:, DIM:2 * DIM])
    n = jnp.tanh(gi[:, 2 * DIM:] + r * gh[:, 2 * DIM:])
    o_ref[...] = (1.0 - z) * n + z * h


def _s2s_body(out_ref, batch_ref, wi_ref, wh_ref, bi_ref, bh_ref, q_ref):
    outv = out_ref[...]
    bI = lax.broadcasted_iota(jnp.int32, (B, N), 0)
    Mb = jnp.broadcast_to(batch_ref[...], (B, N)) == bI
    qs = jnp.zeros((B, 2 * DIM), jnp.float32)
    hs = jnp.zeros((B, DIM), jnp.float32)
    cs = jnp.zeros((B, DIM), jnp.float32)
    for _ in range(3):
        g = (jnp.dot(qs, wi_ref[...], preferred_element_type=jnp.float32)
             + bi_ref[...]
             + jnp.dot(hs, wh_ref[...], preferred_element_type=jnp.float32)
             + bh_ref[...])
        ig = jax.nn.sigmoid(g[:, :DIM])
        fg = jax.nn.sigmoid(g[:, DIM:2 * DIM])
        gg = jnp.tanh(g[:, 2 * DIM:3 * DIM])
        og = jax.nn.sigmoid(g[:, 3 * DIM:])
        cs = fg * cs + ig * gg
        hs = og * jnp.tanh(cs)
        sT = lax.dot_general(hs, outv, (((1,), (1,)), ((), ())),
                             preferred_element_type=jnp.float32)
        emax = jnp.max(jnp.where(Mb, sT, -1e30), axis=1, keepdims=True)
        a = jnp.where(Mb, jnp.exp(sT - emax), 0.0)
        denom = jnp.maximum(jnp.sum(a, axis=1, keepdims=True), 1e-30)
        an = a / denom
        r = jnp.dot(an, outv, preferred_element_type=jnp.float32)
        qs = jnp.concatenate([hs, r], axis=1)
    q_ref[...] = qs


@functools.cache
def _sc_kernels():
    mesh = plsc.VectorSubcoreMesh(core_axis_name="c", subcore_axis_name="s",
                                  num_cores=NC, num_subcores=NS)
    params = pltpu.CompilerParams(use_tc_tiling_on_sc=False)

    @functools.partial(
        pl.kernel, mesh=mesh, compiler_params=params,
        out_type=jax.ShapeDtypeStruct((EPAD, DIM), jnp.float32),
        scratch_types=[
            pltpu.VMEM((NCHUNK_W, CH), jnp.int32),
            pltpu.VMEM((2, CH, DIM), jnp.float32),
            pltpu.SemaphoreType.DMA,
        ])
    def sc_gather(h_hbm, src_hbm, xj_hbm, idx_v, rows_v, sem):
        w = lax.axis_index("s") * NC + lax.axis_index("c")
        pltpu.sync_copy(src_hbm.at[pl.ds(w * NCHUNK_W, NCHUNK_W)], idx_v)
        pltpu.async_copy(h_hbm.at[idx_v.at[0]], rows_v.at[0], sem)

        def body(j, carry):
            @pl.when(j + 1 < NCHUNK_W)
            def _():
                pltpu.async_copy(h_hbm.at[idx_v.at[j + 1]],
                                 rows_v.at[(j + 1) % 2], sem)
            pltpu.make_async_copy(h_hbm.at[idx_v.at[j]],
                                  rows_v.at[j % 2], sem).wait()
            pltpu.sync_copy(rows_v.at[j % 2],
                            xj_hbm.at[pl.ds(w * ROWS_W + j * CH, CH)])
            return carry

        lax.fori_loop(0, NCHUNK_W, body, 0)

    @functools.partial(
        pl.kernel, mesh=mesh, compiler_params=params,
        out_type=jax.ShapeDtypeStruct((NC, N, WIDTH), jnp.float32),
        scratch_types=[
            pltpu.VMEM((NCHUNK_W, CH), jnp.int32),
            pltpu.VMEM((2, CH, WIDTH), jnp.float32),
            pltpu.VMEM_SHARED((N, WIDTH), jnp.float32),
            pltpu.SemaphoreType.DMA,
        ])
    def sc_scatter(msg_hbm, dst_hbm, zeros_hbm, out_hbm, idx_v, vals_v,
                   acc_sh, sem):
        c = lax.axis_index("c")
        s = lax.axis_index("s")
        w = s * NC + c
        pltpu.sync_copy(zeros_hbm, acc_sh.at[pl.ds(s * ROWS_S, ROWS_S)])
        plsc.subcore_barrier()
        pltpu.sync_copy(dst_hbm.at[pl.ds(w * NCHUNK_W, NCHUNK_W)], idx_v)
        pltpu.async_copy(msg_hbm.at[pl.ds(w * ROWS_W, CH)], vals_v.at[0],
                         sem)

        def body(j, carry):
            @pl.when(j + 1 < NCHUNK_W)
            def _():
                pltpu.async_copy(
                    msg_hbm.at[pl.ds(w * ROWS_W + (j + 1) * CH, CH)],
                    vals_v.at[(j + 1) % 2], sem)
            pltpu.make_async_copy(
                msg_hbm.at[pl.ds(w * ROWS_W + j * CH, CH)],
                vals_v.at[j % 2], sem).wait()
            pltpu.sync_copy(vals_v.at[j % 2], acc_sh.at[idx_v.at[j]],
                            add=True)
            return carry

        lax.fori_loop(0, NCHUNK_W, body, 0)
        plsc.subcore_barrier()
        pltpu.sync_copy(acc_sh.at[pl.ds(s * ROWS_S, ROWS_S)],
                        out_hbm.at[c, pl.ds(s * ROWS_S, ROWS_S)])

    return sc_gather, sc_scatter


def _tc(body, grid, in_specs, out_specs, out_shape):
    return pl.pallas_call(body, grid=grid, in_specs=in_specs,
                          out_specs=out_specs, out_shape=out_shape)


def _pad_half(v, width=None):
    pad = EPAD - EH
    if width is None:
        z = jnp.zeros((pad,), v.dtype)
        return jnp.concatenate([v, z]).reshape(EPAD // CH, CH)
    z = jnp.zeros((pad, width), v.dtype)
    return jnp.concatenate([v, z])


def kernel(x, edge_index, edge_attr, batch, W0, b0, We1, be1, We2, be2,
           bconv, Wih, Whh, bih, bhh, Ws_ih, Ws_hh, bs_ih, bs_hh):
    f32 = jnp.float32
    src = edge_index[0]
    dst = edge_index[1]
    srcs = [_pad_half(src[:EH]), _pad_half(src[EH:])]
    dsts = [_pad_half(dst[:EH]), _pad_half(dst[EH:])]
    eas = [_pad_half(edge_attr[:EH], D_EDGE),
           _pad_half(edge_attr[EH:], D_EDGE)]

    cidx = jnp.arange(DIM * DIM, dtype=jnp.int32)
    R = (cidx[None, :] // DIM
         == jnp.arange(DIM, dtype=jnp.int32)[:, None]).astype(jnp.bfloat16)
    S = (cidx[:, None] % DIM
         == jnp.arange(DIM, dtype=jnp.int32)[None, :]).astype(jnp.bfloat16)
    zeros625 = jnp.zeros((ROWS_S, WIDTH), f32)

    full = lambda shp: pl.BlockSpec(shp, lambda: (0,) * len(shp))

    h = _tc(_in_mlp_body, (NBLK_N,),
            [pl.BlockSpec((N // NBLK_N, F_IN), lambda i: (i, 0)),
             pl.BlockSpec((F_IN, DIM), lambda i: (0, 0)),
             pl.BlockSpec((1, DIM), lambda i: (0, 0))],
            pl.BlockSpec((N // NBLK_N, DIM), lambda i: (i, 0)),
            jax.ShapeDtypeStruct((N, DIM), f32))(x, W0.T, b0[None])

    ew_call = _tc(_ew_body, (NBLK_E,),
                  [pl.BlockSpec((EBLK, D_EDGE), lambda i: (i, 0)),
                   pl.BlockSpec((D_EDGE, F_IN), lambda i: (0, 0)),
                   pl.BlockSpec((1, F_IN), lambda i: (0, 0)),
                   pl.BlockSpec((F_IN, DIM * DIM), lambda i: (0, 0)),
                   pl.BlockSpec((1, DIM * DIM), lambda i: (0, 0))],
                  pl.BlockSpec((EBLK, DIM * DIM), lambda i: (i, 0)),
                  jax.ShapeDtypeStruct((EPAD, DIM * DIM), jnp.bfloat16))
    We1T = We1.T
    We2T = We2.T.astype(jnp.bfloat16)
    ews = [ew_call(eas[0], We1T, be1[None], We2T, be2[None]),
           ew_call(eas[1], We1T, be1[None], We2T, be2[None])]

    msg_call = _tc(_msg_body, (NBLK_E,),
                   [pl.BlockSpec((EBLK, DIM), lambda i: (i, 0)),
                    pl.BlockSpec((EBLK, DIM * DIM), lambda i: (i, 0)),
                    pl.BlockSpec((DIM, DIM * DIM), lambda i: (0, 0)),
                    pl.BlockSpec((DIM * DIM, DIM), lambda i: (0, 0))],
                   pl.BlockSpec((EBLK, WIDTH), lambda i: (i, 0)),
                   jax.ShapeDtypeStruct((EPAD, WIDTH), f32))

    gru_call = _tc(_gru_body, (NBLK_N,),
                   [pl.BlockSpec((NC, N // NBLK_N, WIDTH),
                                 lambda i: (0, i, 0)),
                    pl.BlockSpec((NC, N // NBLK_N, WIDTH),
                                 lambda i: (0, i, 0)),
                    pl.BlockSpec((N // NBLK_N, DIM), lambda i: (i, 0)),
                    pl.BlockSpec((DIM, 3 * DIM), lambda i: (0, 0)),
                    pl.BlockSpec((DIM, 3 * DIM), lambda i: (0, 0)),
                    pl.BlockSpec((1, 3 * DIM), lambda i: (0, 0)),
                    pl.BlockSpec((1, 3 * DIM), lambda i: (0, 0)),
                    pl.BlockSpec((1, DIM), lambda i: (0, 0))],
                   pl.BlockSpec((N // NBLK_N, DIM), lambda i: (i, 0)),
                   jax.ShapeDtypeStruct((N, DIM), f32))

    sc_gather, sc_scatter = _sc_kernels()
    WihT = Wih.T
    WhhT = Whh.T
    for _ in range(3):
        xj0 = sc_gather(h, srcs[0])
        msg0 = msg_call(xj0, ews[0], R, S)
        xj1 = sc_gather(h, srcs[1])
        parts0 = sc_scatter(msg0, dsts[0], zeros625)
        msg1 = msg_call(xj1, ews[1], R, S)
        parts1 = sc_scatter(msg1, dsts[1], zeros625)
        h = gru_call(parts0, parts1, h, WihT, WhhT, bih[None], bhh[None],
                     bconv[None])

    qstar = _tc(_s2s_body, (),
                [full((N, DIM)), full((1, N)),
                 full((2 * DIM, 4 * DIM)), full((DIM, 4 * DIM)),
                 full((1, 4 * DIM)), full((1, 4 * DIM))],
                full((B, 2 * DIM)),
                jax.ShapeDtypeStruct((B, 2 * DIM), f32))(
                    h, batch[None], Ws_ih.T, Ws_hh.T, bs_ih[None],
                    bs_hh[None])
    return (qstar, h)


# speedup vs baseline: 1.2586x; 1.0514x over previous
import functools

import jax
import jax.numpy as jnp
from jax import lax
from jax.experimental import pallas as pl
from jax.experimental.pallas import tpu as pltpu
from jax.experimental.pallas import tpu_sc as plsc

N = 10000
E = 160000
F_IN = 128
DIM = 32
B = 128
D_EDGE = 5

NC = 2
NS = 16
NW = NC * NS
CH = 128
EH = E // 2
NCHUNK_W = 20
ROWS_W = CH * NCHUNK_W
EPAD = NW * ROWS_W
WIDTH = DIM + 16
EBLK = 4096
NBLK_E = EPAD // EBLK
NBLK_N = 10
ROWS_S = N // NS


def _in_mlp_body(x_ref, w_ref, b_ref, o_ref):
    o_ref[...] = jax.nn.relu(
        jnp.dot(x_ref[...], w_ref[...], preferred_element_type=jnp.float32)
        + b_ref[...])


def _ew_body(ea_ref, w1_ref, b1_ref, w2_ref, b2_ref, o_ref):
    t = jax.nn.relu(
        jnp.dot(ea_ref[...], w1_ref[...], preferred_element_type=jnp.float32)
        + b1_ref[...]).astype(jnp.bfloat16)
    o_ref[...] = (jnp.dot(t, w2_ref[...], preferred_element_type=jnp.float32)
                  + b2_ref[...]).astype(jnp.bfloat16)


def _msg_body(xj_ref, ew_ref, r_ref, s_ref, o_ref):
    i = pl.program_id(0)
    xe = jnp.dot(xj_ref[...].astype(jnp.bfloat16), r_ref[...],
                 preferred_element_type=jnp.float32)
    p = xe.astype(jnp.bfloat16) * ew_ref[...]
    msg = jnp.dot(p, s_ref[...], preferred_element_type=jnp.float32)
    rows = i * EBLK + lax.broadcasted_iota(jnp.int32, (EBLK, 1), 0)
    m = (rows < EH).astype(jnp.float32)
    o_ref[...] = jnp.concatenate(
        [msg * m, jnp.broadcast_to(m, (EBLK, 16))], axis=1)


def _gru_body(pa_ref, pb_ref, h_ref, wih_ref, whh_ref, bih_ref, bhh_ref,
              bc_ref, o_ref):
    s = pa_ref[0] + pa_ref[1] + pb_ref[0] + pb_ref[1]
    ssum = s[:, :DIM]
    cnt = s[:, DIM:DIM + 1]
    h = h_ref[...]
    aggr = ssum / jnp.maximum(cnt, 1.0) + bc_ref[...]
    m = jax.nn.relu(aggr)
    gi = jnp.dot(m, wih_ref[...], preferred_element_type=jnp.float32) \
        + bih_ref[...]
    gh = jnp.dot(h, whh_ref[...], preferred_element_type=jnp.float32) \
        + bhh_ref[...]
    r = jax.nn.sigmoid(gi[:, :DIM] + gh[:, :DIM])
    z = jax.nn.sigmoid(gi[:, DIM:2 * DIM] + gh[:, DIM:2 * DIM])
    n = jnp.tanh(gi[:, 2 * DIM:] + r * gh[:, 2 * DIM:])
    o_ref[...] = (1.0 - z) * n + z * h


def _s2s_body(out_ref, batch_ref, wi_ref, wh_ref, bi_ref, bh_ref, q_ref):
    outv = out_ref[...]
    bI = lax.broadcasted_iota(jnp.int32, (B, N), 0)
    Mb = jnp.broadcast_to(batch_ref[...], (B, N)) == bI
    qs = jnp.zeros((B, 2 * DIM), jnp.float32)
    hs = jnp.zeros((B, DIM), jnp.float32)
    cs = jnp.zeros((B, DIM), jnp.float32)
    for _ in range(3):
        g = (jnp.dot(qs, wi_ref[...], preferred_element_type=jnp.float32)
             + bi_ref[...]
             + jnp.dot(hs, wh_ref[...], preferred_element_type=jnp.float32)
             + bh_ref[...])
        ig = jax.nn.sigmoid(g[:, :DIM])
        fg = jax.nn.sigmoid(g[:, DIM:2 * DIM])
        gg = jnp.tanh(g[:, 2 * DIM:3 * DIM])
        og = jax.nn.sigmoid(g[:, 3 * DIM:])
        cs = fg * cs + ig * gg
        hs = og * jnp.tanh(cs)
        sT = lax.dot_general(hs, outv, (((1,), (1,)), ((), ())),
                             preferred_element_type=jnp.float32)
        emax = jnp.max(jnp.where(Mb, sT, -1e30), axis=1, keepdims=True)
        a = jnp.where(Mb, jnp.exp(sT - emax), 0.0)
        denom = jnp.maximum(jnp.sum(a, axis=1, keepdims=True), 1e-30)
        an = a / denom
        r = jnp.dot(an, outv, preferred_element_type=jnp.float32)
        qs = jnp.concatenate([hs, r], axis=1)
    q_ref[...] = qs


@functools.cache
def _sc_kernels():
    mesh = plsc.VectorSubcoreMesh(core_axis_name="c", subcore_axis_name="s",
                                  num_cores=NC, num_subcores=NS)
    params = pltpu.CompilerParams(use_tc_tiling_on_sc=False)

    @functools.partial(
        pl.kernel, mesh=mesh, compiler_params=params,
        out_type=jax.ShapeDtypeStruct((EPAD, DIM), jnp.float32),
        scratch_types=[
            pltpu.VMEM((NCHUNK_W, CH), jnp.int32),
            pltpu.VMEM((2, CH, DIM), jnp.float32),
            pltpu.SemaphoreType.DMA,
        ])
    def sc_gather(h_hbm, src_hbm, xj_hbm, idx_v, rows_v, sem):
        w = lax.axis_index("s") * NC + lax.axis_index("c")
        pltpu.sync_copy(src_hbm.at[pl.ds(w * NCHUNK_W, NCHUNK_W)], idx_v)
        pltpu.async_copy(h_hbm.at[idx_v.at[0]], rows_v.at[0], sem)

        def body(j, carry):
            @pl.when(j + 1 < NCHUNK_W)
            def _():
                pltpu.async_copy(h_hbm.at[idx_v.at[j + 1]],
                                 rows_v.at[(j + 1) % 2], sem)
            pltpu.make_async_copy(h_hbm.at[idx_v.at[j]],
                                  rows_v.at[j % 2], sem).wait()
            pltpu.sync_copy(rows_v.at[j % 2],
                            xj_hbm.at[pl.ds(w * ROWS_W + j * CH, CH)])
            return carry

        lax.fori_loop(0, NCHUNK_W, body, 0)

    @functools.partial(
        pl.kernel, mesh=mesh, compiler_params=params,
        out_type=jax.ShapeDtypeStruct((NC, N, WIDTH), jnp.float32),
        scratch_types=[
            pltpu.VMEM((NCHUNK_W, CH), jnp.int32),
            pltpu.VMEM((2, CH, WIDTH), jnp.float32),
            pltpu.VMEM_SHARED((N, WIDTH), jnp.float32),
            pltpu.SemaphoreType.DMA,
        ])
    def sc_scatter(msg_hbm, dst_hbm, zeros_hbm, out_hbm, idx_v, vals_v,
                   acc_sh, sem):
        c = lax.axis_index("c")
        s = lax.axis_index("s")
        w = s * NC + c
        pltpu.sync_copy(zeros_hbm, acc_sh.at[pl.ds(s * ROWS_S, ROWS_S)])
        plsc.subcore_barrier()
        pltpu.sync_copy(dst_hbm.at[pl.ds(w * NCHUNK_W, NCHUNK_W)], idx_v)
        pltpu.async_copy(msg_hbm.at[pl.ds(w * ROWS_W, CH)], vals_v.at[0],
                         sem)

        def body(j, carry):
            @pl.when(j + 1 < NCHUNK_W)
            def _():
                pltpu.async_copy(
                    msg_hbm.at[pl.ds(w * ROWS_W + (j + 1) * CH, CH)],
                    vals_v.at[(j + 1) % 2], sem)
            pltpu.make_async_copy(
                msg_hbm.at[pl.ds(w * ROWS_W + j * CH, CH)],
                vals_v.at[j % 2], sem).wait()
            pltpu.sync_copy(vals_v.at[j % 2], acc_sh.at[idx_v.at[j]],
                            add=True)
            return carry

        lax.fori_loop(0, NCHUNK_W, body, 0)
        plsc.subcore_barrier()
        pltpu.sync_copy(acc_sh.at[pl.ds(s * ROWS_S, ROWS_S)],
                        out_hbm.at[c, pl.ds(s * ROWS_S, ROWS_S)])

    return sc_gather, sc_scatter


def _tc(body, grid, in_specs, out_specs, out_shape):
    return pl.pallas_call(body, grid=grid, in_specs=in_specs,
                          out_specs=out_specs, out_shape=out_shape)


def _pad_half(v, width=None):
    pad = EPAD - EH
    if width is None:
        z = jnp.zeros((pad,), v.dtype)
        return jnp.concatenate([v, z]).reshape(EPAD // CH, CH)
    z = jnp.zeros((pad, width), v.dtype)
    return jnp.concatenate([v, z])


def kernel(x, edge_index, edge_attr, batch, W0, b0, We1, be1, We2, be2,
           bconv, Wih, Whh, bih, bhh, Ws_ih, Ws_hh, bs_ih, bs_hh):
    f32 = jnp.float32
    src = edge_index[0]
    dst = edge_index[1]
    srcs = [_pad_half(src[:EH]), _pad_half(src[EH:])]
    dsts = [_pad_half(dst[:EH]), _pad_half(dst[EH:])]
    eas = [_pad_half(edge_attr[:EH], D_EDGE),
           _pad_half(edge_attr[EH:], D_EDGE)]

    cidx = jnp.arange(DIM * DIM, dtype=jnp.int32)
    R = (cidx[None, :] // DIM
         == jnp.arange(DIM, dtype=jnp.int32)[:, None]).astype(jnp.bfloat16)
    S = (cidx[:, None] % DIM
         == jnp.arange(DIM, dtype=jnp.int32)[None, :]).astype(jnp.bfloat16)
    zeros625 = jnp.zeros((ROWS_S, WIDTH), f32)

    full = lambda shp: pl.BlockSpec(shp, lambda: (0,) * len(shp))

    h = _tc(_in_mlp_body, (NBLK_N,),
            [pl.BlockSpec((N // NBLK_N, F_IN), lambda i: (i, 0)),
             pl.BlockSpec((F_IN, DIM), lambda i: (0, 0)),
             pl.BlockSpec((1, DIM), lambda i: (0, 0))],
            pl.BlockSpec((N // NBLK_N, DIM), lambda i: (i, 0)),
            jax.ShapeDtypeStruct((N, DIM), f32))(x, W0.T, b0[None])

    ew_call = _tc(_ew_body, (NBLK_E,),
                  [pl.BlockSpec((EBLK, D_EDGE), lambda i: (i, 0)),
                   pl.BlockSpec((D_EDGE, F_IN), lambda i: (0, 0)),
                   pl.BlockSpec((1, F_IN), lambda i: (0, 0)),
                   pl.BlockSpec((F_IN, DIM * DIM), lambda i: (0, 0)),
                   pl.BlockSpec((1, DIM * DIM), lambda i: (0, 0))],
                  pl.BlockSpec((EBLK, DIM * DIM), lambda i: (i, 0)),
                  jax.ShapeDtypeStruct((EPAD, DIM * DIM), jnp.bfloat16))
    We1T = We1.T
    We2T = We2.T.astype(jnp.bfloat16)
    ews = [ew_call(eas[0], We1T, be1[None], We2T, be2[None]),
           ew_call(eas[1], We1T, be1[None], We2T, be2[None])]

    msg_call = _tc(_msg_body, (NBLK_E,),
                   [pl.BlockSpec((EBLK, DIM), lambda i: (i, 0)),
                    pl.BlockSpec((EBLK, DIM * DIM), lambda i: (i, 0)),
                    pl.BlockSpec((DIM, DIM * DIM), lambda i: (0, 0)),
                    pl.BlockSpec((DIM * DIM, DIM), lambda i: (0, 0))],
                   pl.BlockSpec((EBLK, WIDTH), lambda i: (i, 0)),
                   jax.ShapeDtypeStruct((EPAD, WIDTH), f32))

    gru_call = _tc(_gru_body, (NBLK_N,),
                   [pl.BlockSpec((NC, N // NBLK_N, WIDTH),
                                 lambda i: (0, i, 0)),
                    pl.BlockSpec((NC, N // NBLK_N, WIDTH),
                                 lambda i: (0, i, 0)),
                    pl.BlockSpec((N // NBLK_N, DIM), lambda i: (i, 0)),
                    pl.BlockSpec((DIM, 3 * DIM), lambda i: (0, 0)),
                    pl.BlockSpec((DIM, 3 * DIM), lambda i: (0, 0)),
                    pl.BlockSpec((1, 3 * DIM), lambda i: (0, 0)),
                    pl.BlockSpec((1, 3 * DIM), lambda i: (0, 0)),
                    pl.BlockSpec((1, DIM), lambda i: (0, 0))],
                   pl.BlockSpec((N // NBLK_N, DIM), lambda i: (i, 0)),
                   jax.ShapeDtypeStruct((N, DIM), f32))

    sc_gather, sc_scatter = _sc_kernels()
    WihT = Wih.T
    WhhT = Whh.T
    for _ in range(3):
        xj0 = sc_gather(h, srcs[0])
        msg0 = msg_call(xj0, ews[0], R, S)
        xj1 = sc_gather(h, srcs[1])
        parts0 = sc_scatter(msg0, dsts[0], zeros625)
        msg1 = msg_call(xj1, ews[1], R, S)
        parts1 = sc_scatter(msg1, dsts[1], zeros625)
        h = gru_call(parts0, parts1, h, WihT, WhhT, bih[None], bhh[None],
                     bconv[None])

    qstar = _tc(_s2s_body, (),
                [full((N, DIM)), full((1, N)),
                 full((2 * DIM, 4 * DIM)), full((DIM, 4 * DIM)),
                 full((1, 4 * DIM)), full((1, 4 * DIM))],
                full((B, 2 * DIM)),
                jax.ShapeDtypeStruct((B, 2 * DIM), f32))(
                    h, batch[None], Ws_ih.T, Ws_hh.T, bs_ih[None],
                    bs_hh[None])
    return (qstar, h)


# speedup vs baseline: 1.2637x; 1.0040x over previous
import functools

import jax
import jax.numpy as jnp
from jax import lax
from jax.experimental import pallas as pl
from jax.experimental.pallas import tpu as pltpu
from jax.experimental.pallas import tpu_sc as plsc

N = 10000
E = 160000
F_IN = 128
DIM = 32
B = 128
D_EDGE = 5

NC = 2
NS = 16
NW = NC * NS
CH = 128
EH = E // 2
NCHUNK_W = 20
ROWS_W = CH * NCHUNK_W
EPAD = NW * ROWS_W
WIDTH = DIM + 16
EBLK = 4096
NBLK_E = EPAD // EBLK
NBLK_N = 2
ROWS_S = N // NS


def _in_mlp_body(x_ref, w_ref, b_ref, o_ref):
    o_ref[...] = jax.nn.relu(
        jnp.dot(x_ref[...], w_ref[...], preferred_element_type=jnp.float32)
        + b_ref[...])


def _ew_body(ea_ref, w1_ref, b1_ref, w2_ref, b2_ref, o_ref):
    t = jax.nn.relu(
        jnp.dot(ea_ref[...], w1_ref[...], preferred_element_type=jnp.float32)
        + b1_ref[...]).astype(jnp.bfloat16)
    o_ref[...] = (jnp.dot(t, w2_ref[...], preferred_element_type=jnp.float32)
                  + b2_ref[...]).astype(jnp.bfloat16)


def _msg_body(xj_ref, ew_ref, r_ref, s_ref, o_ref):
    i = pl.program_id(0)
    xe = jnp.dot(xj_ref[...].astype(jnp.bfloat16), r_ref[...],
                 preferred_element_type=jnp.float32)
    p = xe.astype(jnp.bfloat16) * ew_ref[...]
    msg = jnp.dot(p, s_ref[...], preferred_element_type=jnp.float32)
    rows = i * EBLK + lax.broadcasted_iota(jnp.int32, (EBLK, 1), 0)
    m = (rows < EH).astype(jnp.float32)
    o_ref[...] = jnp.concatenate(
        [msg * m, jnp.broadcast_to(m, (EBLK, 16))], axis=1)


def _gru_body(pa_ref, pb_ref, h_ref, wih_ref, whh_ref, bih_ref, bhh_ref,
              bc_ref, o_ref):
    s = pa_ref[0] + pa_ref[1] + pb_ref[0] + pb_ref[1]
    ssum = s[:, :DIM]
    cnt = s[:, DIM:DIM + 1]
    h = h_ref[...]
    aggr = ssum / jnp.maximum(cnt, 1.0) + bc_ref[...]
    m = jax.nn.relu(aggr)
    gi = jnp.dot(m, wih_ref[...], preferred_element_type=jnp.float32) \
        + bih_ref[...]
    gh = jnp.dot(h, whh_ref[...], preferred_element_type=jnp.float32) \
        + bhh_ref[...]
    r = jax.nn.sigmoid(gi[:, :DIM] + gh[:, :DIM])
    z = jax.nn.sigmoid(gi[:, DIM:2 * DIM] + gh[:, DIM:2 * DIM])
    n = jnp.tanh(gi[:, 2 * DIM:] + r * gh[:, 2 * DIM:])
    o_ref[...] = (1.0 - z) * n + z * h


def _s2s_body(out_ref, batch_ref, wi_ref, wh_ref, bi_ref, bh_ref, q_ref):
    outv = out_ref[...]
    bI = lax.broadcasted_iota(jnp.int32, (B, N), 0)
    Mb = jnp.broadcast_to(batch_ref[...], (B, N)) == bI
    qs = jnp.zeros((B, 2 * DIM), jnp.float32)
    hs = jnp.zeros((B, DIM), jnp.float32)
    cs = jnp.zeros((B, DIM), jnp.float32)
    for _ in range(3):
        g = (jnp.dot(qs, wi_ref[...], preferred_element_type=jnp.float32)
             + bi_ref[...]
             + jnp.dot(hs, wh_ref[...], preferred_element_type=jnp.float32)
             + bh_ref[...])
        ig = jax.nn.sigmoid(g[:, :DIM])
        fg = jax.nn.sigmoid(g[:, DIM:2 * DIM])
        gg = jnp.tanh(g[:, 2 * DIM:3 * DIM])
        og = jax.nn.sigmoid(g[:, 3 * DIM:])
        cs = fg * cs + ig * gg
        hs = og * jnp.tanh(cs)
        sT = lax.dot_general(hs, outv, (((1,), (1,)), ((), ())),
                             preferred_element_type=jnp.float32)
        emax = jnp.max(jnp.where(Mb, sT, -1e30), axis=1, keepdims=True)
        a = jnp.where(Mb, jnp.exp(sT - emax), 0.0)
        denom = jnp.maximum(jnp.sum(a, axis=1, keepdims=True), 1e-30)
        an = a / denom
        r = jnp.dot(an, outv, preferred_element_type=jnp.float32)
        qs = jnp.concatenate([hs, r], axis=1)
    q_ref[...] = qs


@functools.cache
def _sc_kernels():
    mesh = plsc.VectorSubcoreMesh(core_axis_name="c", subcore_axis_name="s",
                                  num_cores=NC, num_subcores=NS)
    params = pltpu.CompilerParams(use_tc_tiling_on_sc=False)

    @functools.partial(
        pl.kernel, mesh=mesh, compiler_params=params,
        out_type=jax.ShapeDtypeStruct((EPAD, DIM), jnp.float32),
        scratch_types=[
            pltpu.VMEM((NCHUNK_W, CH), jnp.int32),
            pltpu.VMEM((2, CH, DIM), jnp.float32),
            pltpu.SemaphoreType.DMA,
        ])
    def sc_gather(h_hbm, src_hbm, xj_hbm, idx_v, rows_v, sem):
        w = lax.axis_index("s") * NC + lax.axis_index("c")
        pltpu.sync_copy(src_hbm.at[pl.ds(w * NCHUNK_W, NCHUNK_W)], idx_v)
        pltpu.async_copy(h_hbm.at[idx_v.at[0]], rows_v.at[0], sem)

        def body(j, carry):
            @pl.when(j + 1 < NCHUNK_W)
            def _():
                pltpu.async_copy(h_hbm.at[idx_v.at[j + 1]],
                                 rows_v.at[(j + 1) % 2], sem)
            pltpu.make_async_copy(h_hbm.at[idx_v.at[j]],
                                  rows_v.at[j % 2], sem).wait()
            pltpu.sync_copy(rows_v.at[j % 2],
                            xj_hbm.at[pl.ds(w * ROWS_W + j * CH, CH)])
            return carry

        lax.fori_loop(0, NCHUNK_W, body, 0)

    @functools.partial(
        pl.kernel, mesh=mesh, compiler_params=params,
        out_type=jax.ShapeDtypeStruct((NC, N, WIDTH), jnp.float32),
        scratch_types=[
            pltpu.VMEM((NCHUNK_W, CH), jnp.int32),
            pltpu.VMEM((2, CH, WIDTH), jnp.float32),
            pltpu.VMEM_SHARED((N, WIDTH), jnp.float32),
            pltpu.SemaphoreType.DMA,
        ])
    def sc_scatter(msg_hbm, dst_hbm, zeros_hbm, out_hbm, idx_v, vals_v,
                   acc_sh, sem):
        c = lax.axis_index("c")
        s = lax.axis_index("s")
        w = s * NC + c
        pltpu.sync_copy(zeros_hbm, acc_sh.at[pl.ds(s * ROWS_S, ROWS_S)])
        plsc.subcore_barrier()
        pltpu.sync_copy(dst_hbm.at[pl.ds(w * NCHUNK_W, NCHUNK_W)], idx_v)
        pltpu.async_copy(msg_hbm.at[pl.ds(w * ROWS_W, CH)], vals_v.at[0],
                         sem)

        def body(j, carry):
            @pl.when(j + 1 < NCHUNK_W)
            def _():
                pltpu.async_copy(
                    msg_hbm.at[pl.ds(w * ROWS_W + (j + 1) * CH, CH)],
                    vals_v.at[(j + 1) % 2], sem)
            pltpu.make_async_copy(
                msg_hbm.at[pl.ds(w * ROWS_W + j * CH, CH)],
                vals_v.at[j % 2], sem).wait()
            pltpu.sync_copy(vals_v.at[j % 2], acc_sh.at[idx_v.at[j]],
                            add=True)
            return carry

        lax.fori_loop(0, NCHUNK_W, body, 0)
        plsc.subcore_barrier()
        pltpu.sync_copy(acc_sh.at[pl.ds(s * ROWS_S, ROWS_S)],
                        out_hbm.at[c, pl.ds(s * ROWS_S, ROWS_S)])

    return sc_gather, sc_scatter


def _tc(body, grid, in_specs, out_specs, out_shape):
    return pl.pallas_call(body, grid=grid, in_specs=in_specs,
                          out_specs=out_specs, out_shape=out_shape)


def _pad_half(v, width=None):
    pad = EPAD - EH
    if width is None:
        z = jnp.zeros((pad,), v.dtype)
        return jnp.concatenate([v, z]).reshape(EPAD // CH, CH)
    z = jnp.zeros((pad, width), v.dtype)
    return jnp.concatenate([v, z])


def kernel(x, edge_index, edge_attr, batch, W0, b0, We1, be1, We2, be2,
           bconv, Wih, Whh, bih, bhh, Ws_ih, Ws_hh, bs_ih, bs_hh):
    f32 = jnp.float32
    src = edge_index[0]
    dst = edge_index[1]
    srcs = [_pad_half(src[:EH]), _pad_half(src[EH:])]
    dsts = [_pad_half(dst[:EH]), _pad_half(dst[EH:])]
    eas = [_pad_half(edge_attr[:EH], D_EDGE),
           _pad_half(edge_attr[EH:], D_EDGE)]

    cidx = jnp.arange(DIM * DIM, dtype=jnp.int32)
    R = (cidx[None, :] // DIM
         == jnp.arange(DIM, dtype=jnp.int32)[:, None]).astype(jnp.bfloat16)
    S = (cidx[:, None] % DIM
         == jnp.arange(DIM, dtype=jnp.int32)[None, :]).astype(jnp.bfloat16)
    zeros625 = jnp.zeros((ROWS_S, WIDTH), f32)

    full = lambda shp: pl.BlockSpec(shp, lambda: (0,) * len(shp))

    h = _tc(_in_mlp_body, (NBLK_N,),
            [pl.BlockSpec((N // NBLK_N, F_IN), lambda i: (i, 0)),
             pl.BlockSpec((F_IN, DIM), lambda i: (0, 0)),
             pl.BlockSpec((1, DIM), lambda i: (0, 0))],
            pl.BlockSpec((N // NBLK_N, DIM), lambda i: (i, 0)),
            jax.ShapeDtypeStruct((N, DIM), f32))(x, W0.T, b0[None])

    ew_call = _tc(_ew_body, (NBLK_E,),
                  [pl.BlockSpec((EBLK, D_EDGE), lambda i: (i, 0)),
                   pl.BlockSpec((D_EDGE, F_IN), lambda i: (0, 0)),
                   pl.BlockSpec((1, F_IN), lambda i: (0, 0)),
                   pl.BlockSpec((F_IN, DIM * DIM), lambda i: (0, 0)),
                   pl.BlockSpec((1, DIM * DIM), lambda i: (0, 0))],
                  pl.BlockSpec((EBLK, DIM * DIM), lambda i: (i, 0)),
                  jax.ShapeDtypeStruct((EPAD, DIM * DIM), jnp.bfloat16))
    We1T = We1.T
    We2T = We2.T.astype(jnp.bfloat16)
    ews = [ew_call(eas[0], We1T, be1[None], We2T, be2[None]),
           ew_call(eas[1], We1T, be1[None], We2T, be2[None])]

    msg_call = _tc(_msg_body, (NBLK_E,),
                   [pl.BlockSpec((EBLK, DIM), lambda i: (i, 0)),
                    pl.BlockSpec((EBLK, DIM * DIM), lambda i: (i, 0)),
                    pl.BlockSpec((DIM, DIM * DIM), lambda i: (0, 0)),
                    pl.BlockSpec((DIM * DIM, DIM), lambda i: (0, 0))],
                   pl.BlockSpec((EBLK, WIDTH), lambda i: (i, 0)),
                   jax.ShapeDtypeStruct((EPAD, WIDTH), f32))

    gru_call = _tc(_gru_body, (NBLK_N,),
                   [pl.BlockSpec((NC, N // NBLK_N, WIDTH),
                                 lambda i: (0, i, 0)),
                    pl.BlockSpec((NC, N // NBLK_N, WIDTH),
                                 lambda i: (0, i, 0)),
                    pl.BlockSpec((N // NBLK_N, DIM), lambda i: (i, 0)),
                    pl.BlockSpec((DIM, 3 * DIM), lambda i: (0, 0)),
                    pl.BlockSpec((DIM, 3 * DIM), lambda i: (0, 0)),
                    pl.BlockSpec((1, 3 * DIM), lambda i: (0, 0)),
                    pl.BlockSpec((1, 3 * DIM), lambda i: (0, 0)),
                    pl.BlockSpec((1, DIM), lambda i: (0, 0))],
                   pl.BlockSpec((N // NBLK_N, DIM), lambda i: (i, 0)),
                   jax.ShapeDtypeStruct((N, DIM), f32))

    sc_gather, sc_scatter = _sc_kernels()
    WihT = Wih.T
    WhhT = Whh.T
    for _ in range(3):
        xj0 = sc_gather(h, srcs[0])
        msg0 = msg_call(xj0, ews[0], R, S)
        xj1 = sc_gather(h, srcs[1])
        parts0 = sc_scatter(msg0, dsts[0], zeros625)
        msg1 = msg_call(xj1, ews[1], R, S)
        parts1 = sc_scatter(msg1, dsts[1], zeros625)
        h = gru_call(parts0, parts1, h, WihT, WhhT, bih[None], bhh[None],
                     bconv[None])

    qstar = _tc(_s2s_body, (),
                [full((N, DIM)), full((1, N)),
                 full((2 * DIM, 4 * DIM)), full((DIM, 4 * DIM)),
                 full((1, 4 * DIM)), full((1, 4 * DIM))],
                full((B, 2 * DIM)),
                jax.ShapeDtypeStruct((B, 2 * DIM), f32))(
                    h, batch[None], Ws_ih.T, Ws_hh.T, bs_ih[None],
                    bs_hh[None])
    return (qstar, h)


# speedup vs baseline: 1.2769x; 1.0105x over previous
import functools

import jax
import jax.numpy as jnp
from jax import lax
from jax.experimental import pallas as pl
from jax.experimental.pallas import tpu as pltpu
from jax.experimental.pallas import tpu_sc as plsc

N = 10000
E = 160000
F_IN = 128
DIM = 32
B = 128
D_EDGE = 5

NC = 2
NS = 16
NW = NC * NS
CH = 128
EH = E
NCHUNK_W = 40
ROWS_W = CH * NCHUNK_W
EPAD = NW * ROWS_W
WIDTH = DIM + 16
EBLK = 4096
NBLK_E = EPAD // EBLK
NBLK_N = 2
ROWS_S = N // NS


def _in_mlp_body(x_ref, w_ref, b_ref, o_ref):
    o_ref[...] = jax.nn.relu(
        jnp.dot(x_ref[...], w_ref[...], preferred_element_type=jnp.float32)
        + b_ref[...])


def _ew_body(ea_ref, w1_ref, b1_ref, w2_ref, b2_ref, o_ref):
    t = jax.nn.relu(
        jnp.dot(ea_ref[...], w1_ref[...], preferred_element_type=jnp.float32)
        + b1_ref[...]).astype(jnp.bfloat16)
    o_ref[...] = (jnp.dot(t, w2_ref[...], preferred_element_type=jnp.float32)
                  + b2_ref[...]).astype(jnp.bfloat16)


def _msg_body(xj_ref, ew_ref, r_ref, s_ref, o_ref):
    i = pl.program_id(0)
    xe = jnp.dot(xj_ref[...].astype(jnp.bfloat16), r_ref[...],
                 preferred_element_type=jnp.float32)
    p = xe.astype(jnp.bfloat16) * ew_ref[...]
    msg = jnp.dot(p, s_ref[...], preferred_element_type=jnp.float32)
    rows = i * EBLK + lax.broadcasted_iota(jnp.int32, (EBLK, 1), 0)
    m = (rows < EH).astype(jnp.float32)
    o_ref[...] = jnp.concatenate(
        [msg * m, jnp.broadcast_to(m, (EBLK, 16))], axis=1)


def _gru_body(pa_ref, h_ref, wih_ref, whh_ref, bih_ref, bhh_ref,
              bc_ref, o_ref):
    s = pa_ref[0] + pa_ref[1]
    ssum = s[:, :DIM]
    cnt = s[:, DIM:DIM + 1]
    h = h_ref[...]
    aggr = ssum / jnp.maximum(cnt, 1.0) + bc_ref[...]
    m = jax.nn.relu(aggr)
    gi = jnp.dot(m, wih_ref[...], preferred_element_type=jnp.float32) \
        + bih_ref[...]
    gh = jnp.dot(h, whh_ref[...], preferred_element_type=jnp.float32) \
        + bhh_ref[...]
    r = jax.nn.sigmoid(gi[:, :DIM] + gh[:, :DIM])
    z = jax.nn.sigmoid(gi[:, DIM:2 * DIM] + gh[:, DIM:2 * DIM])
    n = jnp.tanh(gi[:, 2 * DIM:] + r * gh[:, 2 * DIM:])
    o_ref[...] = (1.0 - z) * n + z * h


def _s2s_body(out_ref, batch_ref, wi_ref, wh_ref, bi_ref, bh_ref, q_ref):
    outv = out_ref[...]
    bI = lax.broadcasted_iota(jnp.int32, (B, N), 0)
    Mb = jnp.broadcast_to(batch_ref[...], (B, N)) == bI
    qs = jnp.zeros((B, 2 * DIM), jnp.float32)
    hs = jnp.zeros((B, DIM), jnp.float32)
    cs = jnp.zeros((B, DIM), jnp.float32)
    for _ in range(3):
        g = (jnp.dot(qs, wi_ref[...], preferred_element_type=jnp.float32)
             + bi_ref[...]
             + jnp.dot(hs, wh_ref[...], preferred_element_type=jnp.float32)
             + bh_ref[...])
        ig = jax.nn.sigmoid(g[:, :DIM])
        fg = jax.nn.sigmoid(g[:, DIM:2 * DIM])
        gg = jnp.tanh(g[:, 2 * DIM:3 * DIM])
        og = jax.nn.sigmoid(g[:, 3 * DIM:])
        cs = fg * cs + ig * gg
        hs = og * jnp.tanh(cs)
        sT = lax.dot_general(hs, outv, (((1,), (1,)), ((), ())),
                             preferred_element_type=jnp.float32)
        emax = jnp.max(jnp.where(Mb, sT, -1e30), axis=1, keepdims=True)
        a = jnp.where(Mb, jnp.exp(sT - emax), 0.0)
        denom = jnp.maximum(jnp.sum(a, axis=1, keepdims=True), 1e-30)
        an = a / denom
        r = jnp.dot(an, outv, preferred_element_type=jnp.float32)
        qs = jnp.concatenate([hs, r], axis=1)
    q_ref[...] = qs


@functools.cache
def _sc_kernels():
    mesh = plsc.VectorSubcoreMesh(core_axis_name="c", subcore_axis_name="s",
                                  num_cores=NC, num_subcores=NS)
    params = pltpu.CompilerParams(use_tc_tiling_on_sc=False)

    @functools.partial(
        pl.kernel, mesh=mesh, compiler_params=params,
        out_type=jax.ShapeDtypeStruct((EPAD, DIM), jnp.float32),
        scratch_types=[
            pltpu.VMEM((NCHUNK_W, CH), jnp.int32),
            pltpu.VMEM((2, CH, DIM), jnp.float32),
            pltpu.SemaphoreType.DMA,
        ])
    def sc_gather(h_hbm, src_hbm, xj_hbm, idx_v, rows_v, sem):
        w = lax.axis_index("s") * NC + lax.axis_index("c")
        pltpu.sync_copy(src_hbm.at[pl.ds(w * NCHUNK_W, NCHUNK_W)], idx_v)
        pltpu.async_copy(h_hbm.at[idx_v.at[0]], rows_v.at[0], sem)

        def body(j, carry):
            @pl.when(j + 1 < NCHUNK_W)
            def _():
                pltpu.async_copy(h_hbm.at[idx_v.at[j + 1]],
                                 rows_v.at[(j + 1) % 2], sem)
            pltpu.make_async_copy(h_hbm.at[idx_v.at[j]],
                                  rows_v.at[j % 2], sem).wait()
            pltpu.sync_copy(rows_v.at[j % 2],
                            xj_hbm.at[pl.ds(w * ROWS_W + j * CH, CH)])
            return carry

        lax.fori_loop(0, NCHUNK_W, body, 0)

    @functools.partial(
        pl.kernel, mesh=mesh, compiler_params=params,
        out_type=jax.ShapeDtypeStruct((NC, N, WIDTH), jnp.float32),
        scratch_types=[
            pltpu.VMEM((NCHUNK_W, CH), jnp.int32),
            pltpu.VMEM((2, CH, WIDTH), jnp.float32),
            pltpu.VMEM_SHARED((N, WIDTH), jnp.float32),
            pltpu.SemaphoreType.DMA,
        ])
    def sc_scatter(msg_hbm, dst_hbm, zeros_hbm, out_hbm, idx_v, vals_v,
                   acc_sh, sem):
        c = lax.axis_index("c")
        s = lax.axis_index("s")
        w = s * NC + c
        pltpu.sync_copy(zeros_hbm, acc_sh.at[pl.ds(s * ROWS_S, ROWS_S)])
        plsc.subcore_barrier()
        pltpu.sync_copy(dst_hbm.at[pl.ds(w * NCHUNK_W, NCHUNK_W)], idx_v)
        pltpu.async_copy(msg_hbm.at[pl.ds(w * ROWS_W, CH)], vals_v.at[0],
                         sem)

        def body(j, carry):
            @pl.when(j + 1 < NCHUNK_W)
            def _():
                pltpu.async_copy(
                    msg_hbm.at[pl.ds(w * ROWS_W + (j + 1) * CH, CH)],
                    vals_v.at[(j + 1) % 2], sem)
            pltpu.make_async_copy(
                msg_hbm.at[pl.ds(w * ROWS_W + j * CH, CH)],
                vals_v.at[j % 2], sem).wait()
            pltpu.sync_copy(vals_v.at[j % 2], acc_sh.at[idx_v.at[j]],
                            add=True)
            return carry

        lax.fori_loop(0, NCHUNK_W, body, 0)
        plsc.subcore_barrier()
        pltpu.sync_copy(acc_sh.at[pl.ds(s * ROWS_S, ROWS_S)],
                        out_hbm.at[c, pl.ds(s * ROWS_S, ROWS_S)])

    return sc_gather, sc_scatter


def _tc(body, grid, in_specs, out_specs, out_shape):
    return pl.pallas_call(body, grid=grid, in_specs=in_specs,
                          out_specs=out_specs, out_shape=out_shape)


def _pad_half(v, width=None):
    pad = EPAD - EH
    if width is None:
        z = jnp.zeros((pad,), v.dtype)
        return jnp.concatenate([v, z]).reshape(EPAD // CH, CH)
    z = jnp.zeros((pad, width), v.dtype)
    return jnp.concatenate([v, z])


def kernel(x, edge_index, edge_attr, batch, W0, b0, We1, be1, We2, be2,
           bconv, Wih, Whh, bih, bhh, Ws_ih, Ws_hh, bs_ih, bs_hh):
    f32 = jnp.float32
    src = edge_index[0]
    dst = edge_index[1]
    src_p = _pad_half(src)
    dst_p = _pad_half(dst)
    ea_p = _pad_half(edge_attr, D_EDGE)

    cidx = jnp.arange(DIM * DIM, dtype=jnp.int32)
    R = (cidx[None, :] // DIM
         == jnp.arange(DIM, dtype=jnp.int32)[:, None]).astype(jnp.bfloat16)
    S = (cidx[:, None] % DIM
         == jnp.arange(DIM, dtype=jnp.int32)[None, :]).astype(jnp.bfloat16)
    zeros625 = jnp.zeros((ROWS_S, WIDTH), f32)

    full = lambda shp: pl.BlockSpec(shp, lambda: (0,) * len(shp))

    h = _tc(_in_mlp_body, (NBLK_N,),
            [pl.BlockSpec((N // NBLK_N, F_IN), lambda i: (i, 0)),
             pl.BlockSpec((F_IN, DIM), lambda i: (0, 0)),
             pl.BlockSpec((1, DIM), lambda i: (0, 0))],
            pl.BlockSpec((N // NBLK_N, DIM), lambda i: (i, 0)),
            jax.ShapeDtypeStruct((N, DIM), f32))(x, W0.T, b0[None])

    ew_call = _tc(_ew_body, (NBLK_E,),
                  [pl.BlockSpec((EBLK, D_EDGE), lambda i: (i, 0)),
                   pl.BlockSpec((D_EDGE, F_IN), lambda i: (0, 0)),
                   pl.BlockSpec((1, F_IN), lambda i: (0, 0)),
                   pl.BlockSpec((F_IN, DIM * DIM), lambda i: (0, 0)),
                   pl.BlockSpec((1, DIM * DIM), lambda i: (0, 0))],
                  pl.BlockSpec((EBLK, DIM * DIM), lambda i: (i, 0)),
                  jax.ShapeDtypeStruct((EPAD, DIM * DIM), jnp.bfloat16))
    ew = ew_call(ea_p, We1.T, be1[None], We2.T.astype(jnp.bfloat16),
                 be2[None])

    msg_call = _tc(_msg_body, (NBLK_E,),
                   [pl.BlockSpec((EBLK, DIM), lambda i: (i, 0)),
                    pl.BlockSpec((EBLK, DIM * DIM), lambda i: (i, 0)),
                    pl.BlockSpec((DIM, DIM * DIM), lambda i: (0, 0)),
                    pl.BlockSpec((DIM * DIM, DIM), lambda i: (0, 0))],
                   pl.BlockSpec((EBLK, WIDTH), lambda i: (i, 0)),
                   jax.ShapeDtypeStruct((EPAD, WIDTH), f32))

    gru_call = _tc(_gru_body, (NBLK_N,),
                   [pl.BlockSpec((NC, N // NBLK_N, WIDTH),
                                 lambda i: (0, i, 0)),
                    pl.BlockSpec((N // NBLK_N, DIM), lambda i: (i, 0)),
                    pl.BlockSpec((DIM, 3 * DIM), lambda i: (0, 0)),
                    pl.BlockSpec((DIM, 3 * DIM), lambda i: (0, 0)),
                    pl.BlockSpec((1, 3 * DIM), lambda i: (0, 0)),
                    pl.BlockSpec((1, 3 * DIM), lambda i: (0, 0)),
                    pl.BlockSpec((1, DIM), lambda i: (0, 0))],
                   pl.BlockSpec((N // NBLK_N, DIM), lambda i: (i, 0)),
                   jax.ShapeDtypeStruct((N, DIM), f32))

    sc_gather, sc_scatter = _sc_kernels()
    WihT = Wih.T
    WhhT = Whh.T
    for _ in range(3):
        xj = sc_gather(h, src_p)
        msg = msg_call(xj, ew, R, S)
        parts = sc_scatter(msg, dst_p, zeros625)
        h = gru_call(parts, h, WihT, WhhT, bih[None], bhh[None],
                     bconv[None])

    qstar = _tc(_s2s_body, (),
                [full((N, DIM)), full((1, N)),
                 full((2 * DIM, 4 * DIM)), full((DIM, 4 * DIM)),
                 full((1, 4 * DIM)), full((1, 4 * DIM))],
                full((B, 2 * DIM)),
                jax.ShapeDtypeStruct((B, 2 * DIM), f32))(
                    h, batch[None], Ws_ih.T, Ws_hh.T, bs_ih[None],
                    bs_hh[None])
    return (qstar, h)
